# Initial kernel scaffold; baseline (speedup 1.0000x reference)
import jax
import jax.numpy as jnp
from jax.experimental import pallas as pl


def kernel(nodes, item_history, itemrating_history, social_history, user_table, item_table, rating_table, i_ln1_w, i_ln1_b, i_ln2_w, i_ln2_b, i_ln3_w, i_ln3_b, ia1_w, ia1_b, ia2_w, ia2_b, ia3_w, ia3_b, s_ln1_w, s_ln1_b, s_ln2_w, s_ln2_b, s_ln3_w, s_ln3_b, sa1_w, sa1_b, sa2_w, sa2_b, sa3_w, sa3_b):
    raise NotImplementedError("write your pallas kernel here")


# speedup vs baseline: 3.6289x; 3.6289x over previous
import functools

import jax
import jax.numpy as jnp
from jax import lax
from jax.experimental import pallas as pl
from jax.experimental.pallas import tpu as pltpu
from jax.experimental.pallas import tpu_sc as plsc

_NC = 2
_NS = 16
_NW = _NC * _NS


def _gather_rows(table, idx, chunk):
    V, Dp = table.shape
    N = idx.shape[0]
    assert N % _NW == 0
    n_w = N // _NW
    assert n_w % chunk == 0 and chunk % 8 == 0 and chunk <= 128
    steps = n_w // chunk
    mesh = plsc.VectorSubcoreMesh(core_axis_name="c", subcore_axis_name="s")

    @functools.partial(
        pl.kernel,
        mesh=mesh,
        compiler_params=pltpu.CompilerParams(use_tc_tiling_on_sc=False),
        out_type=jax.ShapeDtypeStruct((N, Dp), table.dtype),
        scratch_types=[
            pltpu.VMEM((chunk,), jnp.int32),
            pltpu.VMEM((chunk, Dp), table.dtype),
            pltpu.SemaphoreType.DMA,
        ],
    )
    def k(table_hbm, idx_hbm, out_hbm, idx_v, rows_v, sem):
        wid = lax.axis_index("s") * _NC + lax.axis_index("c")

        def body(s, carry):
            base = wid * n_w + s * chunk
            pltpu.sync_copy(idx_hbm.at[pl.ds(base, chunk)], idx_v)
            pltpu.async_copy(table_hbm.at[idx_v], rows_v, sem).wait()
            pltpu.sync_copy(rows_v, out_hbm.at[pl.ds(base, chunk)])
            return carry

        lax.fori_loop(0, steps, body, 0)

    return k(table, idx)


def _stage_a_body(wi_ref, rat_ref, wu_ref, rt_ref,
                  w1a_ref, w1b_ref, b1_ref,
                  a1a_ref, a1b_ref, ab1_ref, a2w_ref, a2b_ref, a3_ref,
                  w2_ref, b2_ref, w3a_ref, w3b_ref, b3_ref, out_ref):
    L = wi_ref.shape[0]
    relu = lambda x: jnp.maximum(x, 0.0)
    mm = lambda a, b: jnp.dot(a, b, preferred_element_type=jnp.float32)
    wu = wu_ref[:]
    rat = rat_ref[:]
    rt = rt_ref[:]
    iota8 = lax.broadcasted_iota(jnp.int32, (1, 8), 1)
    rtw1b = mm(rt, w1b_ref[:])
    u_att = mm(wu, a1b_ref[:]) + ab1_ref[:]
    b1 = b1_ref[:]
    a2b = a2b_ref[:]
    a3 = a3_ref[:]
    xs = []
    ls = []
    for j in range(L):
        wi_j = wi_ref[j]
        oh_j = (rat[:, j:j + 1] == iota8).astype(jnp.float32)
        x_j = relu(mm(wi_j, w1a_ref[:]) + mm(oh_j, rtw1b) + b1)
        h = relu(mm(x_j, a1a_ref[:]) + u_att)
        h = relu(mm(h, a2w_ref[:]) + a2b)
        l_j = jnp.sum(h * a3, axis=1, keepdims=True)
        xs.append(x_j)
        ls.append(l_j)
    m = ls[0]
    for j in range(1, L):
        m = jnp.maximum(m, ls[j])
    es = [jnp.exp(l_j - m) for l_j in ls]
    denom = es[0]
    for j in range(1, L):
        denom = denom + es[j]
    acc = xs[0] * es[0]
    for j in range(1, L):
        acc = acc + xs[j] * es[j]
    hI = acc / denom
    hI = relu(mm(hI, w2_ref[:]) + b2_ref[:])
    out_ref[:] = relu(mm(wu, w3a_ref[:]) + mm(hI, w3b_ref[:]) + b3_ref[:])


def _stage_a(witem3, ratings, wuser, rt_pad, w1a, w1b, b1,
             a1a, a1b, ab1, a2w, a2b, a3row, w2, b2, w3a, w3b, b3):
    L, N, D = witem3.shape
    BU = 256
    grid = N // BU
    full = lambda arr: pl.BlockSpec(arr.shape, lambda i: (0,) * arr.ndim)
    return pl.pallas_call(
        _stage_a_body,
        grid=(grid,),
        in_specs=[
            pl.BlockSpec((L, BU, D), lambda i: (0, i, 0)),
            pl.BlockSpec((BU, L), lambda i: (i, 0)),
            pl.BlockSpec((BU, D), lambda i: (i, 0)),
            full(rt_pad), full(w1a), full(w1b), full(b1),
            full(a1a), full(a1b), full(ab1), full(a2w), full(a2b), full(a3row),
            full(w2), full(b2), full(w3a), full(w3b), full(b3),
        ],
        out_specs=pl.BlockSpec((BU, D), lambda i: (i, 0)),
        out_shape=jax.ShapeDtypeStruct((N, D), jnp.float32),
    )(witem3, ratings, wuser, rt_pad, w1a, w1b, b1,
      a1a, a1b, ab1, a2w, a2b, a3row, w2, b2, w3a, w3b, b3)


def _stage_b_body(hs_ref, hi_ref, wu_ref,
                  sa1a_ref, sa1b_ref, sab1_ref, sa2w_ref, sa2b_ref, sa3_ref,
                  s1w_ref, s1b_ref, s2a_ref, s2b_ref, s2bias_ref,
                  s3w_ref, s3b_ref, out_ref):
    S = hs_ref.shape[0]
    relu = lambda x: jnp.maximum(x, 0.0)
    mm = lambda a, b: jnp.dot(a, b, preferred_element_type=jnp.float32)
    wu = wu_ref[:]
    u_att = mm(wu, sa1b_ref[:]) + sab1_ref[:]
    sa2b = sa2b_ref[:]
    sa3 = sa3_ref[:]
    zs = []
    ls = []
    for s in range(S):
        z_s = hs_ref[s]
        a = relu(mm(z_s, sa1a_ref[:]) + u_att)
        a = relu(mm(a, sa2w_ref[:]) + sa2b)
        l_s = jnp.sum(a * sa3, axis=1, keepdims=True)
        zs.append(z_s)
        ls.append(l_s)
    m = ls[0]
    for s in range(1, S):
        m = jnp.maximum(m, ls[s])
    es = [jnp.exp(l_s - m) for l_s in ls]
    denom = es[0]
    for s in range(1, S):
        denom = denom + es[s]
    acc = zs[0] * es[0]
    for s in range(1, S):
        acc = acc + zs[s] * es[s]
    hS = acc / denom
    hS = relu(mm(hS, s1w_ref[:]) + s1b_ref[:])
    f = relu(mm(hi_ref[:], s2a_ref[:]) + mm(hS, s2b_ref[:]) + s2bias_ref[:])
    out_ref[:] = relu(mm(f, s3w_ref[:]) + s3b_ref[:])


def _stage_b(hIs3, hI, wuser, sa1a, sa1b, sab1, sa2w, sa2b, sa3row,
             s1w, s1b, s2a, s2b, s2bias, s3w, s3b):
    S, B, D = hIs3.shape
    BN = 512
    grid = B // BN
    full = lambda arr: pl.BlockSpec(arr.shape, lambda i: (0,) * arr.ndim)
    return pl.pallas_call(
        _stage_b_body,
        grid=(grid,),
        in_specs=[
            pl.BlockSpec((S, BN, D), lambda i: (0, i, 0)),
            pl.BlockSpec((BN, D), lambda i: (i, 0)),
            pl.BlockSpec((BN, D), lambda i: (i, 0)),
            full(sa1a), full(sa1b), full(sab1), full(sa2w), full(sa2b),
            full(sa3row), full(s1w), full(s1b), full(s2a), full(s2b),
            full(s2bias), full(s3w), full(s3b),
        ],
        out_specs=pl.BlockSpec((BN, D), lambda i: (i, 0)),
        out_shape=jax.ShapeDtypeStruct((B, D), jnp.float32),
    )(hIs3, hI, wuser, sa1a, sa1b, sab1, sa2w, sa2b, sa3row,
      s1w, s1b, s2a, s2b, s2bias, s3w, s3b)


def kernel(nodes, item_history, itemrating_history, social_history,
           user_table, item_table, rating_table,
           i_ln1_w, i_ln1_b, i_ln2_w, i_ln2_b, i_ln3_w, i_ln3_b,
           ia1_w, ia1_b, ia2_w, ia2_b, ia3_w, ia3_b,
           s_ln1_w, s_ln1_b, s_ln2_w, s_ln2_b, s_ln3_w, s_ln3_b,
           sa1_w, sa1_b, sa2_w, sa2_b, sa3_w, sa3_b):
    NU, L = item_history.shape
    S = social_history.shape[1]
    D = user_table.shape[1]
    B = nodes.shape[0]
    i32 = jnp.int32
    f32 = jnp.float32
    nodes = nodes.astype(i32)

    hist_cat = jnp.concatenate(
        [item_history.astype(i32), itemrating_history.astype(i32),
         jnp.zeros((NU, 8), i32)], axis=1)
    social_pad = jnp.concatenate(
        [social_history.astype(i32), jnp.zeros((NU, 12), i32)], axis=1)

    soc_g = _gather_rows(social_pad, nodes, 32)
    soc = soc_g[:, :S]
    u_all = jnp.concatenate([nodes, soc.T.reshape(-1)])
    NT = u_all.shape[0]

    hist_g = _gather_rows(hist_cat, u_all, 112)
    wuser_g = _gather_rows(user_table.astype(f32), u_all, 112)

    items_jm = hist_g[:, :L].T.reshape(-1)
    ratings_u = hist_g[:, L:2 * L]
    witem = _gather_rows(item_table.astype(f32), items_jm, 128)
    witem3 = witem.reshape(L, NT, D)

    rt_pad = jnp.concatenate(
        [rating_table.astype(f32),
         jnp.zeros((8 - rating_table.shape[0], D), f32)], axis=0)
    row = lambda v: v.reshape(1, -1).astype(f32)
    feat = _stage_a(
        witem3, ratings_u, wuser_g, rt_pad,
        i_ln1_w[:D], i_ln1_w[D:], row(i_ln1_b),
        ia1_w[:D], ia1_w[D:], row(ia1_b), ia2_w, row(ia2_b), row(ia3_w),
        i_ln2_w, row(i_ln2_b), i_ln3_w[:D], i_ln3_w[D:], row(i_ln3_b))

    hI = feat[:B]
    hIs3 = feat[B:].reshape(S, B, D)
    return _stage_b(
        hIs3, hI, wuser_g[:B],
        sa1_w[:D], sa1_w[D:], row(sa1_b), sa2_w, row(sa2_b), row(sa3_w),
        s_ln1_w, row(s_ln1_b), s_ln2_w[:D], s_ln2_w[D:], row(s_ln2_b),
        s_ln3_w, row(s_ln3_b))


# speedup vs baseline: 3.6460x; 1.0047x over previous
import functools

import jax
import jax.numpy as jnp
from jax import lax
from jax.experimental import pallas as pl
from jax.experimental.pallas import tpu as pltpu
from jax.experimental.pallas import tpu_sc as plsc

_NC = 2
_NS = 16
_NW = _NC * _NS


def _gather_rows(table, idx, chunk):
    V, Dp = table.shape
    N = idx.shape[0]
    assert N % _NW == 0
    n_w = N // _NW
    assert n_w % chunk == 0 and chunk % 8 == 0 and chunk <= 128
    steps = n_w // chunk
    mesh = plsc.VectorSubcoreMesh(core_axis_name="c", subcore_axis_name="s")

    @functools.partial(
        pl.kernel,
        mesh=mesh,
        compiler_params=pltpu.CompilerParams(use_tc_tiling_on_sc=False),
        out_type=jax.ShapeDtypeStruct((N, Dp), table.dtype),
        scratch_types=[
            pltpu.VMEM((chunk,), jnp.int32),
            pltpu.VMEM((chunk, Dp), table.dtype),
            pltpu.SemaphoreType.DMA,
        ],
    )
    def k(table_hbm, idx_hbm, out_hbm, idx_v, rows_v, sem):
        wid = lax.axis_index("s") * _NC + lax.axis_index("c")

        def body(s, carry):
            base = wid * n_w + s * chunk
            pltpu.sync_copy(idx_hbm.at[pl.ds(base, chunk)], idx_v)
            pltpu.async_copy(table_hbm.at[idx_v], rows_v, sem).wait()
            pltpu.sync_copy(rows_v, out_hbm.at[pl.ds(base, chunk)])
            return carry

        lax.fori_loop(0, steps, body, 0)

    return k(table, idx)


def _stage_a_body(wi_ref, rat_ref, wu_ref, rt_ref,
                  w1a_ref, w1b_ref, b1_ref,
                  a1a_ref, a1b_ref, ab1_ref, a2w_ref, a2b_ref, a3_ref,
                  w2_ref, b2_ref, w3a_ref, w3b_ref, b3_ref, out_ref):
    L = wi_ref.shape[0]
    relu = lambda x: jnp.maximum(x, 0.0)
    mm = lambda a, b: jnp.dot(a.astype(jnp.bfloat16), b.astype(jnp.bfloat16),
                              preferred_element_type=jnp.float32)
    wu = wu_ref[:]
    rat = rat_ref[:]
    rt = rt_ref[:]
    iota8 = lax.broadcasted_iota(jnp.int32, (1, 8), 1)
    rtw1b = mm(rt, w1b_ref[:])
    u_att = mm(wu, a1b_ref[:]) + ab1_ref[:]
    b1 = b1_ref[:]
    a2b = a2b_ref[:]
    a3 = a3_ref[:]
    xs = []
    ls = []
    for j in range(L):
        wi_j = wi_ref[j]
        oh_j = (rat[:, j:j + 1] == iota8).astype(jnp.float32)
        x_j = relu(mm(wi_j, w1a_ref[:]) + mm(oh_j, rtw1b) + b1)
        h = relu(mm(x_j, a1a_ref[:]) + u_att)
        h = relu(mm(h, a2w_ref[:]) + a2b)
        l_j = jnp.sum(h * a3, axis=1, keepdims=True)
        xs.append(x_j)
        ls.append(l_j)
    m = ls[0]
    for j in range(1, L):
        m = jnp.maximum(m, ls[j])
    es = [jnp.exp(l_j - m) for l_j in ls]
    denom = es[0]
    for j in range(1, L):
        denom = denom + es[j]
    acc = xs[0] * es[0]
    for j in range(1, L):
        acc = acc + xs[j] * es[j]
    hI = acc / denom
    hI = relu(mm(hI, w2_ref[:]) + b2_ref[:])
    out_ref[:] = relu(mm(wu, w3a_ref[:]) + mm(hI, w3b_ref[:]) + b3_ref[:])


def _stage_a(witem3, ratings, wuser, rt_pad, w1a, w1b, b1,
             a1a, a1b, ab1, a2w, a2b, a3row, w2, b2, w3a, w3b, b3):
    L, N, D = witem3.shape
    BU = 256
    grid = N // BU
    full = lambda arr: pl.BlockSpec(arr.shape, lambda i: (0,) * arr.ndim)
    return pl.pallas_call(
        _stage_a_body,
        grid=(grid,),
        in_specs=[
            pl.BlockSpec((L, BU, D), lambda i: (0, i, 0)),
            pl.BlockSpec((BU, L), lambda i: (i, 0)),
            pl.BlockSpec((BU, D), lambda i: (i, 0)),
            full(rt_pad), full(w1a), full(w1b), full(b1),
            full(a1a), full(a1b), full(ab1), full(a2w), full(a2b), full(a3row),
            full(w2), full(b2), full(w3a), full(w3b), full(b3),
        ],
        out_specs=pl.BlockSpec((BU, D), lambda i: (i, 0)),
        out_shape=jax.ShapeDtypeStruct((N, D), jnp.float32),
    )(witem3, ratings, wuser, rt_pad, w1a, w1b, b1,
      a1a, a1b, ab1, a2w, a2b, a3row, w2, b2, w3a, w3b, b3)


def _stage_b_body(f_ref, wu_ref,
                  sa1a_ref, sa1b_ref, sab1_ref, sa2w_ref, sa2b_ref, sa3_ref,
                  s1w_ref, s1b_ref, s2a_ref, s2b_ref, s2bias_ref,
                  s3w_ref, s3b_ref, out_ref):
    S = f_ref.shape[0] - 1
    relu = lambda x: jnp.maximum(x, 0.0)
    mm = lambda a, b: jnp.dot(a.astype(jnp.bfloat16), b.astype(jnp.bfloat16),
                              preferred_element_type=jnp.float32)
    wu = wu_ref[:]
    u_att = mm(wu, sa1b_ref[:]) + sab1_ref[:]
    sa2b = sa2b_ref[:]
    sa3 = sa3_ref[:]
    zs = []
    ls = []
    for s in range(S):
        z_s = f_ref[s + 1]
        a = relu(mm(z_s, sa1a_ref[:]) + u_att)
        a = relu(mm(a, sa2w_ref[:]) + sa2b)
        l_s = jnp.sum(a * sa3, axis=1, keepdims=True)
        zs.append(z_s)
        ls.append(l_s)
    m = ls[0]
    for s in range(1, S):
        m = jnp.maximum(m, ls[s])
    es = [jnp.exp(l_s - m) for l_s in ls]
    denom = es[0]
    for s in range(1, S):
        denom = denom + es[s]
    acc = zs[0] * es[0]
    for s in range(1, S):
        acc = acc + zs[s] * es[s]
    hS = acc / denom
    hS = relu(mm(hS, s1w_ref[:]) + s1b_ref[:])
    f = relu(mm(f_ref[0], s2a_ref[:]) + mm(hS, s2b_ref[:]) + s2bias_ref[:])
    out_ref[:] = relu(mm(f, s3w_ref[:]) + s3b_ref[:])


def _stage_b(feat3, wuser, sa1a, sa1b, sab1, sa2w, sa2b, sa3row,
             s1w, s1b, s2a, s2b, s2bias, s3w, s3b):
    S1, B, D = feat3.shape
    BN = 512
    grid = B // BN
    full = lambda arr: pl.BlockSpec(arr.shape, lambda i: (0,) * arr.ndim)
    return pl.pallas_call(
        _stage_b_body,
        grid=(grid,),
        in_specs=[
            pl.BlockSpec((S1, BN, D), lambda i: (0, i, 0)),
            pl.BlockSpec((BN, D), lambda i: (i, 0)),
            full(sa1a), full(sa1b), full(sab1), full(sa2w), full(sa2b),
            full(sa3row), full(s1w), full(s1b), full(s2a), full(s2b),
            full(s2bias), full(s3w), full(s3b),
        ],
        out_specs=pl.BlockSpec((BN, D), lambda i: (i, 0)),
        out_shape=jax.ShapeDtypeStruct((B, D), jnp.float32),
    )(feat3, wuser, sa1a, sa1b, sab1, sa2w, sa2b, sa3row,
      s1w, s1b, s2a, s2b, s2bias, s3w, s3b)


def kernel(nodes, item_history, itemrating_history, social_history,
           user_table, item_table, rating_table,
           i_ln1_w, i_ln1_b, i_ln2_w, i_ln2_b, i_ln3_w, i_ln3_b,
           ia1_w, ia1_b, ia2_w, ia2_b, ia3_w, ia3_b,
           s_ln1_w, s_ln1_b, s_ln2_w, s_ln2_b, s_ln3_w, s_ln3_b,
           sa1_w, sa1_b, sa2_w, sa2_b, sa3_w, sa3_b):
    NU, L = item_history.shape
    S = social_history.shape[1]
    D = user_table.shape[1]
    B = nodes.shape[0]
    i32 = jnp.int32
    f32 = jnp.float32
    nodes = nodes.astype(i32)

    hist_cat = jnp.concatenate(
        [item_history.astype(i32), itemrating_history.astype(i32),
         jnp.zeros((NU, 8), i32)], axis=1)
    social_pad = jnp.concatenate(
        [social_history.astype(i32), jnp.zeros((NU, 12), i32)], axis=1)

    soc = _gather_rows(social_pad, nodes, 32)[:, :S]
    u_all = jnp.concatenate([nodes, soc.T.reshape(-1)])
    NT = u_all.shape[0]

    hist_g = _gather_rows(hist_cat, u_all, 112)
    ratings_u = hist_g[:, L:2 * L]
    wuser_g = _gather_rows(user_table.astype(f32), u_all, 112)

    items_jm = hist_g[:, :L].T.reshape(-1)
    witem = _gather_rows(item_table.astype(f32), items_jm, 128)
    witem3 = witem.reshape(L, NT, D)

    rt_pad = jnp.concatenate(
        [rating_table.astype(f32),
         jnp.zeros((8 - rating_table.shape[0], D), f32)], axis=0)
    row = lambda v: v.reshape(1, -1).astype(f32)
    feat = _stage_a(
        witem3, ratings_u, wuser_g, rt_pad,
        i_ln1_w[:D], i_ln1_w[D:], row(i_ln1_b),
        ia1_w[:D], ia1_w[D:], row(ia1_b), ia2_w, row(ia2_b), row(ia3_w),
        i_ln2_w, row(i_ln2_b), i_ln3_w[:D], i_ln3_w[D:], row(i_ln3_b))

    feat3 = feat.reshape(S + 1, B, D)
    return _stage_b(
        feat3, wuser_g,
        sa1_w[:D], sa1_w[D:], row(sa1_b), sa2_w, row(sa2_b), row(sa3_w),
        s_ln1_w, row(s_ln1_b), s_ln2_w[:D], s_ln2_w[D:], row(s_ln2_b),
        s_ln3_w, row(s_ln3_b))
